# Initial kernel scaffold; baseline (speedup 1.0000x reference)
import functools

import jax
import jax.numpy as jnp
from jax import lax
from jax.experimental import pallas as pl
from jax.experimental.pallas import tpu as pltpu
from jax.experimental.pallas import tpu_sc as plsc

N = 10000
E = 320000
DIN = 128
DE = 16
DOUT = 128
DAUG = 144

NC = 2
NS = 16
NW = NC * NS
EPT = E // NW
CH = 125
NCHUNK = EPT // CH
ROWS_PT = N // NS

_mesh = plsc.VectorSubcoreMesh(
    core_axis_name="c", subcore_axis_name="s", num_cores=NC, num_subcores=NS)


@functools.partial(
    pl.kernel,
    out_type=(
        jax.ShapeDtypeStruct((NC, N, DAUG), jnp.float32),
        jax.ShapeDtypeStruct((NC, N, DE), jnp.float32),
    ),
    mesh=_mesh,
    scratch_types=[
        pltpu.VMEM((NCHUNK, CH), jnp.int32),
        pltpu.VMEM((NCHUNK, CH), jnp.int32),
        pltpu.VMEM((CH, DAUG), jnp.float32),
        pltpu.VMEM((CH, DE), jnp.float32),
        pltpu.VMEM_SHARED((N, DAUG), jnp.float32),
        pltpu.VMEM_SHARED((N, DE), jnp.float32),
        pltpu.SemaphoreType.DMA,
    ],
)
def _scatter_kernel(nfa, efeats2, src_r, dst_r, zn, ze,
                    out_n, out_e,
                    idx_s, idx_d, rows, erows, acc_n, acc_e, sem):
    cid = lax.axis_index("c")
    sid = lax.axis_index("s")
    tid = cid * NS + sid

    r0 = sid * ROWS_PT
    pltpu.sync_copy(zn.at[pl.ds(r0, ROWS_PT)], acc_n.at[pl.ds(r0, ROWS_PT)])
    pltpu.sync_copy(ze.at[pl.ds(r0, ROWS_PT)], acc_e.at[pl.ds(r0, ROWS_PT)])
    plsc.subcore_barrier()

    pltpu.sync_copy(src_r.at[tid], idx_s)
    pltpu.sync_copy(dst_r.at[tid], idx_d)

    ebase = tid * EPT

    @pl.loop(0, NCHUNK)
    def _chunk(j):
        pltpu.async_copy(nfa.at[idx_s.at[j]], rows, sem).wait()
        pltpu.sync_copy(efeats2.at[pl.ds(ebase + j * CH, CH)], erows)
        pltpu.sync_copy(rows, acc_n.at[idx_d.at[j]], add=True)
        pltpu.sync_copy(erows, acc_e.at[idx_d.at[j]], add=True)

    plsc.subcore_barrier()

    pltpu.sync_copy(acc_n.at[pl.ds(r0, ROWS_PT)],
                    out_n.at[cid].at[pl.ds(r0, ROWS_PT)])
    pltpu.sync_copy(acc_e.at[pl.ds(r0, ROWS_PT)],
                    out_e.at[cid].at[pl.ds(r0, ROWS_PT)])


BLK = 1000


def _dense_body(snp, sep, nf, wmsg, bmsg, wapp, bapp, nh_out, hh_out):
    sn = snp[0] + snp[1]
    se = sep[0] + sep[1]
    deg = sn[:, DIN:DIN + 1]
    inv = 1.0 / jnp.maximum(deg, 1.0)
    w1 = wmsg[:DIN, :]
    w2 = wmsg[DIN:, :]
    msum = jnp.dot(sn[:, :DIN] * inv, w1, preferred_element_type=jnp.float32)
    msum = msum + jnp.dot(se * inv, w2, preferred_element_type=jnp.float32)
    hn = msum + jnp.where(deg > 0.0, 1.0, 0.0) * bmsg[0]
    wa1 = wapp[:DIN, :]
    wa2 = wapp[DIN:, :]
    pre = (jnp.dot(nf[0], wa1, preferred_element_type=jnp.float32)
           + jnp.dot(hn, wa2, preferred_element_type=jnp.float32)
           + bapp[0])
    nh = jnp.maximum(pre, 0.0)
    nh_out[0] = nh
    hh_out[0] = 0.5 * nh


def _dense(snp, sep, nfeats3, W_msg, b_msg, W_apply, b_apply):
    grid = (N // BLK,)
    return pl.pallas_call(
        _dense_body,
        grid=grid,
        in_specs=[
            pl.BlockSpec((NC, BLK, DAUG), lambda i: (0, i, 0)),
            pl.BlockSpec((NC, BLK, DE), lambda i: (0, i, 0)),
            pl.BlockSpec((1, BLK, DIN), lambda i: (0, i, 0)),
            pl.BlockSpec((DIN + DE, DOUT), lambda i: (0, 0)),
            pl.BlockSpec((1, DOUT), lambda i: (0, 0)),
            pl.BlockSpec((DIN + DOUT, DOUT), lambda i: (0, 0)),
            pl.BlockSpec((1, DOUT), lambda i: (0, 0)),
        ],
        out_specs=[
            pl.BlockSpec((1, BLK, DOUT), lambda i: (0, i, 0)),
            pl.BlockSpec((1, BLK, DOUT), lambda i: (0, i, 0)),
        ],
        out_shape=[
            jax.ShapeDtypeStruct((1, N, DOUT), jnp.float32),
            jax.ShapeDtypeStruct((1, N, DOUT), jnp.float32),
        ],
    )(snp, sep, nfeats3, W_msg, b_msg, W_apply, b_apply)


@functools.partial(
    pl.kernel,
    out_type=jax.ShapeDtypeStruct((E, DOUT), jnp.float32),
    mesh=_mesh,
    scratch_types=[
        pltpu.VMEM((NCHUNK, CH), jnp.int32),
        pltpu.VMEM((NCHUNK, CH), jnp.int32),
        pltpu.VMEM((CH, DOUT), jnp.float32),
        pltpu.VMEM((CH, DOUT), jnp.float32),
        pltpu.SemaphoreType.DMA,
    ],
)
def _edge_kernel(hh, src_r, dst_r, out, idx_s, idx_d, bufa, bufb, sem):
    cid = lax.axis_index("c")
    sid = lax.axis_index("s")
    tid = cid * NS + sid

    pltpu.sync_copy(src_r.at[tid], idx_s)
    pltpu.sync_copy(dst_r.at[tid], idx_d)

    ebase = tid * EPT

    @pl.loop(0, NCHUNK)
    def _chunk(j):
        pltpu.async_copy(hh.at[idx_s.at[j]], bufa, sem).wait()
        pltpu.async_copy(hh.at[idx_d.at[j]], bufb, sem).wait()

        @pl.loop(0, CH)
        def _row(r):
            for u in range(DOUT // 16):
                sl = pl.ds(u * 16, 16)
                bufa[r, sl] = bufa[r, sl] + bufb[r, sl]

        pltpu.sync_copy(bufa, out.at[pl.ds(ebase + j * CH, CH)])


@jax.jit
def kernel(nfeats, efeats, edge_index, W_msg, b_msg, W_apply, b_apply):
    nfeats2 = nfeats.reshape(N, DIN)
    efeats2 = efeats.reshape(E, DE)
    src = edge_index[0].astype(jnp.int32)
    dst = edge_index[1].astype(jnp.int32)
    src_r = src.reshape(NW, NCHUNK, CH)
    dst_r = dst.reshape(NW, NCHUNK, CH)

    nfa = jnp.concatenate(
        [nfeats2,
         jnp.ones((N, 1), jnp.float32),
         jnp.zeros((N, DAUG - DIN - 1), jnp.float32)], axis=1)

    zn = jnp.zeros((N, DAUG), jnp.float32)
    ze = jnp.zeros((N, DE), jnp.float32)

    snp, sep = _scatter_kernel(nfa, efeats2, src_r, dst_r, zn, ze)

    nh3, hh3 = _dense(snp, sep, nfeats2[None], W_msg,
                      b_msg[None], W_apply, b_apply[None])
    new_h = nh3.reshape(N, 1, DOUT)
    hh = hh3.reshape(N, DOUT)

    e_out = _edge_kernel(hh, src_r, dst_r)
    return new_h, e_out.reshape(E, 1, DOUT)


# speedup vs baseline: 3.6350x; 3.6350x over previous
import functools

import jax
import jax.numpy as jnp
from jax import lax
from jax.experimental import pallas as pl
from jax.experimental.pallas import tpu as pltpu
from jax.experimental.pallas import tpu_sc as plsc

N = 10000
E = 320000
DIN = 128
DE = 16
DOUT = 128

NC = 2
NS = 16
NW = NC * NS
NHALF = N // NC
NACC = 5120
RPT_A = NACC // NS

EPS = E // NS
CHA = 80
NCHA = EPS // CHA

EPT = E // NW
CHC = 125
NCHC = EPT // CHC

_mesh = plsc.VectorSubcoreMesh(
    core_axis_name="c", subcore_axis_name="s", num_cores=NC, num_subcores=NS)
_sc_params = pltpu.CompilerParams(
    use_tc_tiling_on_sc=False, needs_layout_passes=False)


@functools.partial(
    pl.kernel,
    out_type=(
        jax.ShapeDtypeStruct((NC, NACC, DIN), jnp.float32),
        jax.ShapeDtypeStruct((NC, NACC, DE), jnp.float32),
        jax.ShapeDtypeStruct((NW, N), jnp.float32),
    ),
    mesh=_mesh,
    scratch_types=[
        pltpu.VMEM((NCHA, CHA), jnp.int32),
        pltpu.VMEM((NCHA, CHA), jnp.int32),
        pltpu.VMEM((NCHA, CHA), jnp.int32),
        pltpu.VMEM((CHA, DIN), jnp.float32),
        pltpu.VMEM((CHA, DE), jnp.float32),
        pltpu.VMEM((N,), jnp.float32),
        pltpu.VMEM_SHARED((NACC, DIN), jnp.float32),
        pltpu.VMEM_SHARED((NACC, DE), jnp.float32),
        pltpu.SemaphoreType.DMA,
    ],
    compiler_params=_sc_params,
)
def _scatter_kernel(nfeats2, efeats2, src_r, dst_r, zn, ze,
                    out_n, out_e, out_deg,
                    idx_s, idx_d, idx_o, rows, erows, deg_l,
                    acc_n, acc_e, sem):
    cid = lax.axis_index("c")
    sid = lax.axis_index("s")

    r0 = sid * RPT_A
    pltpu.sync_copy(zn.at[pl.ds(r0, RPT_A)], acc_n.at[pl.ds(r0, RPT_A)])
    pltpu.sync_copy(ze.at[pl.ds(r0, RPT_A)], acc_e.at[pl.ds(r0, RPT_A)])

    zero16 = jnp.zeros((16,), jnp.float32)

    @pl.loop(0, N // 16)
    def _z(g):
        deg_l[pl.ds(g * 16, 16)] = zero16

    pltpu.sync_copy(src_r.at[sid], idx_s)
    pltpu.sync_copy(dst_r.at[sid], idx_d)

    base = cid * NHALF
    ones16 = jnp.ones((16,), jnp.float32)
    dump16 = jnp.full((16,), NHALF, jnp.int32)

    @pl.loop(0, NCHA)
    def _remap(r):
        for k in range(CHA // 16):
            sl = pl.ds(k * 16, 16)
            d = idx_d[r, sl]
            own = d - base
            msk = (own >= 0) & (own < NHALF)
            idx_o[r, sl] = jnp.where(msk, own, dump16)
            plsc.addupdate_scatter(deg_l, [d], ones16)

    plsc.subcore_barrier()

    ebase = sid * EPS

    @pl.loop(0, NCHA)
    def _chunk(j):
        pltpu.async_copy(nfeats2.at[idx_s.at[j]], rows, sem).wait()
        pltpu.sync_copy(efeats2.at[pl.ds(ebase + j * CHA, CHA)], erows)
        pltpu.sync_copy(rows, acc_n.at[idx_o.at[j]], add=True)
        pltpu.sync_copy(erows, acc_e.at[idx_o.at[j]], add=True)

    plsc.subcore_barrier()

    tid = cid * NS + sid
    pltpu.sync_copy(acc_n.at[pl.ds(r0, RPT_A)],
                    out_n.at[cid].at[pl.ds(r0, RPT_A)])
    pltpu.sync_copy(acc_e.at[pl.ds(r0, RPT_A)],
                    out_e.at[cid].at[pl.ds(r0, RPT_A)])
    pltpu.sync_copy(deg_l, out_deg.at[tid])


BLK = 1000
NBLK_HALF = NHALF // BLK


def _dense_body(snp, sep, degp, nf, wmsg, bmsg, wapp, bapp, nh_out, hh_out):
    sn = snp[0]
    se = sep[0]
    deg = 0.5 * jnp.sum(degp[...], axis=1)[:, None]
    inv = 1.0 / jnp.maximum(deg, 1.0)
    w1 = wmsg[:DIN, :]
    w2 = wmsg[DIN:, :]
    msum = jnp.dot(sn * inv, w1, preferred_element_type=jnp.float32)
    msum = msum + jnp.dot(se * inv, w2, preferred_element_type=jnp.float32)
    hn = msum + jnp.where(deg > 0.0, 1.0, 0.0) * bmsg[0]
    wa1 = wapp[:DIN, :]
    wa2 = wapp[DIN:, :]
    pre = (jnp.dot(nf[0], wa1, preferred_element_type=jnp.float32)
           + jnp.dot(hn, wa2, preferred_element_type=jnp.float32)
           + bapp[0])
    nh = jnp.maximum(pre, 0.0)
    nh_out[0] = nh
    hh_out[0] = 0.5 * nh


def _dense(snp, sep, degt, nfeats3, W_msg, b_msg, W_apply, b_apply):
    grid = (N // BLK,)
    return pl.pallas_call(
        _dense_body,
        grid=grid,
        in_specs=[
            pl.BlockSpec((1, BLK, DIN),
                         lambda i: (i // NBLK_HALF, i % NBLK_HALF, 0)),
            pl.BlockSpec((1, BLK, DE),
                         lambda i: (i // NBLK_HALF, i % NBLK_HALF, 0)),
            pl.BlockSpec((BLK, NW), lambda i: (i, 0)),
            pl.BlockSpec((1, BLK, DIN), lambda i: (0, i, 0)),
            pl.BlockSpec((DIN + DE, DOUT), lambda i: (0, 0)),
            pl.BlockSpec((1, DOUT), lambda i: (0, 0)),
            pl.BlockSpec((DIN + DOUT, DOUT), lambda i: (0, 0)),
            pl.BlockSpec((1, DOUT), lambda i: (0, 0)),
        ],
        out_specs=[
            pl.BlockSpec((1, BLK, DOUT), lambda i: (0, i, 0)),
            pl.BlockSpec((1, BLK, DOUT), lambda i: (0, i, 0)),
        ],
        out_shape=[
            jax.ShapeDtypeStruct((1, N, DOUT), jnp.float32),
            jax.ShapeDtypeStruct((1, N, DOUT), jnp.float32),
        ],
    )(snp, sep, degt, nfeats3, W_msg, b_msg, W_apply, b_apply)


@functools.partial(
    pl.kernel,
    out_type=jax.ShapeDtypeStruct((E, DOUT), jnp.float32),
    mesh=_mesh,
    scratch_types=[
        pltpu.VMEM((NCHC, CHC), jnp.int32),
        pltpu.VMEM((NCHC, CHC), jnp.int32),
        pltpu.VMEM((CHC, DOUT), jnp.float32),
        pltpu.VMEM((CHC, DOUT), jnp.float32),
        pltpu.SemaphoreType.DMA,
    ],
    compiler_params=_sc_params,
)
def _edge_kernel(hh, src_r, dst_r, out, idx_s, idx_d, bufa, bufb, sem):
    cid = lax.axis_index("c")
    sid = lax.axis_index("s")
    tid = cid * NS + sid

    pltpu.sync_copy(src_r.at[tid], idx_s)
    pltpu.sync_copy(dst_r.at[tid], idx_d)

    ebase = tid * EPT

    @pl.loop(0, NCHC)
    def _chunk(j):
        pltpu.async_copy(hh.at[idx_s.at[j]], bufa, sem).wait()
        pltpu.async_copy(hh.at[idx_d.at[j]], bufb, sem).wait()

        @pl.loop(0, CHC)
        def _row(r):
            for u in range(DOUT // 16):
                sl = pl.ds(u * 16, 16)
                bufa[r, sl] = bufa[r, sl] + bufb[r, sl]

        pltpu.sync_copy(bufa, out.at[pl.ds(ebase + j * CHC, CHC)])


@jax.jit
def kernel(nfeats, efeats, edge_index, W_msg, b_msg, W_apply, b_apply):
    nfeats2 = nfeats.reshape(N, DIN)
    efeats2 = efeats.reshape(E, DE)
    src = edge_index[0].astype(jnp.int32)
    dst = edge_index[1].astype(jnp.int32)

    zn = jnp.zeros((NACC, DIN), jnp.float32)
    ze = jnp.zeros((NACC, DE), jnp.float32)

    snp, sep, degp = _scatter_kernel(
        nfeats2, efeats2,
        src.reshape(NS, NCHA, CHA), dst.reshape(NS, NCHA, CHA), zn, ze)

    nh3, hh3 = _dense(snp, sep, degp.T, nfeats2[None], W_msg,
                      b_msg[None], W_apply, b_apply[None])
    new_h = nh3.reshape(N, 1, DOUT)
    hh = hh3.reshape(N, DOUT)

    e_out = _edge_kernel(hh, src.reshape(NW, NCHC, CHC),
                         dst.reshape(NW, NCHC, CHC))
    return new_h, e_out.reshape(E, 1, DOUT)


# speedup vs baseline: 5.8653x; 1.6136x over previous
import functools

import jax
import jax.numpy as jnp
from jax import lax
from jax.experimental import pallas as pl
from jax.experimental.pallas import tpu as pltpu
from jax.experimental.pallas import tpu_sc as plsc

N = 10000
E = 320000
DIN = 128
DE = 16
DOUT = 128

NC = 2
NS = 16
NW = NC * NS
NHALF = N // NC
NACC = 5120
RPT_A = NACC // NS

EPS = E // NS
CHA = 80
NCHA = EPS // CHA
NGRP = NCHA // 5

EPT = E // NW
CHC = 125
NCHC = EPT // CHC

_mesh = plsc.VectorSubcoreMesh(
    core_axis_name="c", subcore_axis_name="s", num_cores=NC, num_subcores=NS)
_sc_params = pltpu.CompilerParams(
    use_tc_tiling_on_sc=False, needs_layout_passes=False)


@functools.partial(
    pl.kernel,
    out_type=(
        jax.ShapeDtypeStruct((NC, NACC, DIN), jnp.float32),
        jax.ShapeDtypeStruct((NC, NACC, DE), jnp.float32),
        jax.ShapeDtypeStruct((NW, NHALF), jnp.float32),
    ),
    mesh=_mesh,
    scratch_types=[
        [pltpu.VMEM((5, CHA), jnp.int32) for _ in range(2)],
        pltpu.VMEM((NCHA, CHA), jnp.int32),
        [pltpu.VMEM((CHA, DIN), jnp.float32) for _ in range(5)],
        [pltpu.VMEM((CHA, DE), jnp.float32) for _ in range(5)],
        pltpu.VMEM((NHALF,), jnp.float32),
        pltpu.VMEM_SHARED((NACC, DIN), jnp.float32),
        pltpu.VMEM_SHARED((NACC, DE), jnp.float32),
        [pltpu.SemaphoreType.DMA for _ in range(2)],
        [pltpu.SemaphoreType.DMA for _ in range(5)],
        [pltpu.SemaphoreType.DMA for _ in range(5)],
        [pltpu.SemaphoreType.DMA for _ in range(5)],
        [pltpu.SemaphoreType.DMA for _ in range(5)],
    ],
    compiler_params=_sc_params,
)
def _scatter_kernel(nfeats2, efeats2, src_r, dst_r, zn, ze,
                    out_n, out_e, out_deg,
                    isr, idx_d, rows, erows, deg_l,
                    acc_n, acc_e, isl, gs, ss, el, es):
    cid = lax.axis_index("c")
    sid = lax.axis_index("s")

    r0 = sid * RPT_A
    pltpu.sync_copy(zn.at[pl.ds(r0, RPT_A)], acc_n.at[pl.ds(r0, RPT_A)])
    pltpu.sync_copy(ze.at[pl.ds(r0, RPT_A)], acc_e.at[pl.ds(r0, RPT_A)])

    zero16 = jnp.zeros((16,), jnp.float32)

    @pl.loop(0, NHALF // 16)
    def _z(g):
        deg_l[pl.ds(g * 16, 16)] = zero16

    pltpu.sync_copy(dst_r.at[sid], idx_d)
    base = cid * NHALF
    ones16 = jnp.ones((16,), jnp.float32)
    dump16 = jnp.full((16,), NHALF, jnp.int32)
    zero16i = jnp.zeros((16,), jnp.int32)

    @pl.loop(0, NCHA)
    def _remap(r):
        for k in range(CHA // 16):
            sl = pl.ds(k * 16, 16)
            d = idx_d[r, sl]
            own = d - base
            msk = (own >= 0) & (own < NHALF)
            safe = jnp.where(msk, own, zero16i)
            plsc.addupdate_scatter(deg_l, [safe],
                                   jnp.where(msk, ones16, 0.0))
            idx_d[r, sl] = jnp.where(msk, own, dump16)

    plsc.subcore_barrier()

    ebase = sid * EPS

    def isl_start(g, e):
        pltpu.async_copy(src_r.at[sid].at[pl.ds(g * 5, 5)], isr[e], isl[e])

    def isl_wait(e):
        pltpu.make_async_copy(src_r.at[sid].at[pl.ds(0, 5)],
                              isr[e], isl[e]).wait()

    def g_start(row_ref, b):
        pltpu.async_copy(nfeats2.at[row_ref], rows[b], gs[b])

    def g_wait(b):
        pltpu.make_async_copy(nfeats2.at[isr[0].at[0]], rows[b],
                              gs[b]).wait()

    def e_start(j, b):
        pltpu.async_copy(efeats2.at[pl.ds(ebase + j * CHA, CHA)],
                         erows[b], el[b])

    def e_wait(b):
        pltpu.make_async_copy(efeats2.at[pl.ds(0, CHA)],
                              erows[b], el[b]).wait()

    def s_start(j, b):
        pltpu.async_copy(rows[b], acc_n.at[idx_d.at[j]], ss[b], add=True)

    def s_drain(b):
        pltpu.make_async_copy(rows[b], acc_n.at[idx_d.at[0]], ss[b]).wait()

    def es_start(j, b):
        pltpu.async_copy(erows[b], acc_e.at[idx_d.at[j]], es[b], add=True)

    def es_drain(b):
        pltpu.make_async_copy(erows[b], acc_e.at[idx_d.at[0]],
                              es[b]).wait()

    isl_start(0, 0)
    isl_start(1, 1)
    isl_wait(0)
    for b in range(3):
        g_start(isr[0].at[b], b)
        e_start(b, b)

    @pl.loop(0, NGRP // 2)
    def _dgrp(G):
        for e in range(2):
            g = G * 2 + e
            if e == 0:
                isl_wait(1)
            else:
                @pl.when(G < NGRP // 2 - 1)
                def _():
                    isl_wait(0)
            for b in range(5):
                j = g * 5 + b
                g_wait(b)
                e_wait(b)
                s_start(j, b)
                es_start(j, b)
                b3 = (b + 3) % 5
                @pl.when((j >= 2) & (j + 3 < NCHA))
                def _():
                    s_drain(b3)
                    es_drain(b3)

                @pl.when(j + 3 < NCHA)
                def _():
                    if b < 2:
                        g_start(isr[e].at[b + 3], b3)
                    else:
                        g_start(isr[1 - e].at[b - 2], b3)
                    e_start(j + 3, b3)
            @pl.when(g + 2 < NGRP)
            def _():
                isl_start(g + 2, e)

    for b in range(5):
        s_drain(b)
        es_drain(b)

    plsc.subcore_barrier()

    tid = cid * NS + sid
    pltpu.sync_copy(acc_n.at[pl.ds(r0, RPT_A)],
                    out_n.at[cid].at[pl.ds(r0, RPT_A)])
    pltpu.sync_copy(acc_e.at[pl.ds(r0, RPT_A)],
                    out_e.at[cid].at[pl.ds(r0, RPT_A)])
    pltpu.sync_copy(deg_l, out_deg.at[tid])


BLK = 1000
NBLK_HALF = NHALF // BLK


def _dense_body(snp, sep, degp, nf, wmsg, bmsg, wapp, bapp, nh_out, hh_out):
    sn = snp[0]
    se = sep[0]
    dall = degp[...]
    deg0 = jnp.sum(dall[:, :NS], axis=1)
    deg1 = jnp.sum(dall[:, NS:], axis=1)
    pid = pl.program_id(0)
    deg = jnp.where(pid < NBLK_HALF, deg0, deg1)[:, None]
    inv = 1.0 / jnp.maximum(deg, 1.0)
    w1 = wmsg[:DIN, :]
    w2 = wmsg[DIN:, :]
    msum = jnp.dot(sn * inv, w1, preferred_element_type=jnp.float32)
    msum = msum + jnp.dot(se * inv, w2, preferred_element_type=jnp.float32)
    hn = msum + jnp.where(deg > 0.0, 1.0, 0.0) * bmsg[0]
    wa1 = wapp[:DIN, :]
    wa2 = wapp[DIN:, :]
    pre = (jnp.dot(nf[0], wa1, preferred_element_type=jnp.float32)
           + jnp.dot(hn, wa2, preferred_element_type=jnp.float32)
           + bapp[0])
    nh = jnp.maximum(pre, 0.0)
    nh_out[0] = nh
    hh_out[0] = 0.5 * nh


def _dense(snp, sep, degt, nfeats3, W_msg, b_msg, W_apply, b_apply):
    grid = (N // BLK,)
    return pl.pallas_call(
        _dense_body,
        grid=grid,
        in_specs=[
            pl.BlockSpec((1, BLK, DIN),
                         lambda i: (i // NBLK_HALF, i % NBLK_HALF, 0)),
            pl.BlockSpec((1, BLK, DE),
                         lambda i: (i // NBLK_HALF, i % NBLK_HALF, 0)),
            pl.BlockSpec((BLK, NW), lambda i: (i % NBLK_HALF, 0)),
            pl.BlockSpec((1, BLK, DIN), lambda i: (0, i, 0)),
            pl.BlockSpec((DIN + DE, DOUT), lambda i: (0, 0)),
            pl.BlockSpec((1, DOUT), lambda i: (0, 0)),
            pl.BlockSpec((DIN + DOUT, DOUT), lambda i: (0, 0)),
            pl.BlockSpec((1, DOUT), lambda i: (0, 0)),
        ],
        out_specs=[
            pl.BlockSpec((1, BLK, DOUT), lambda i: (0, i, 0)),
            pl.BlockSpec((1, BLK, DOUT), lambda i: (0, i, 0)),
        ],
        out_shape=[
            jax.ShapeDtypeStruct((1, N, DOUT), jnp.float32),
            jax.ShapeDtypeStruct((1, N, DOUT), jnp.float32),
        ],
    )(snp, sep, degt, nfeats3, W_msg, b_msg, W_apply, b_apply)


@functools.partial(
    pl.kernel,
    out_type=jax.ShapeDtypeStruct((E, DOUT), jnp.float32),
    mesh=_mesh,
    scratch_types=[
        pltpu.VMEM((NCHC, CHC), jnp.int32),
        pltpu.VMEM((NCHC, CHC), jnp.int32),
        [pltpu.VMEM((CHC, DOUT), jnp.float32) for _ in range(2)],
        [pltpu.VMEM((CHC, DOUT), jnp.float32) for _ in range(2)],
        [pltpu.VMEM((CHC, DOUT), jnp.float32) for _ in range(2)],
        [pltpu.SemaphoreType.DMA for _ in range(2)],
        [pltpu.SemaphoreType.DMA for _ in range(2)],
        [pltpu.SemaphoreType.DMA for _ in range(2)],
    ],
    compiler_params=_sc_params,
)
def _edge_kernel(hh, src_r, dst_r, out,
                 idx_s, idx_d, bufa, bufb, bufo, ga, gb, os):
    cid = lax.axis_index("c")
    sid = lax.axis_index("s")
    tid = cid * NS + sid

    pltpu.sync_copy(src_r.at[tid], idx_s)
    pltpu.sync_copy(dst_r.at[tid], idx_d)

    ebase = tid * EPT

    def ga_start(j, b):
        pltpu.async_copy(hh.at[idx_s.at[j]], bufa[b], ga[b])

    def ga_wait(b):
        pltpu.make_async_copy(hh.at[idx_s.at[0]], bufa[b], ga[b]).wait()

    def gb_start(j, b):
        pltpu.async_copy(hh.at[idx_d.at[j]], bufb[b], gb[b])

    def gb_wait(b):
        pltpu.make_async_copy(hh.at[idx_d.at[0]], bufb[b], gb[b]).wait()

    def o_start(j, b):
        pltpu.async_copy(bufo[b], out.at[pl.ds(ebase + j * CHC, CHC)],
                         os[b])

    def o_wait(b):
        pltpu.make_async_copy(bufo[b],
                              out.at[pl.ds(ebase, CHC)], os[b]).wait()

    for b in range(2):
        ga_start(b, b)
        gb_start(b, b)

    @pl.loop(0, NCHC // 2)
    def _grp(G):
        for b in range(2):
            j = G * 2 + b
            ga_wait(b)
            gb_wait(b)

            @pl.when(j >= 2)
            def _():
                o_wait(b)

            @pl.loop(0, CHC)
            def _row(r):
                for u in range(DOUT // 16):
                    sl = pl.ds(u * 16, 16)
                    bufo[b][r, sl] = bufa[b][r, sl] + bufb[b][r, sl]

            @pl.when(j + 2 < NCHC)
            def _():
                ga_start(j + 2, b)
                gb_start(j + 2, b)

            o_start(j, b)

    for b in range(2):
        o_wait(b)


@jax.jit
def kernel(nfeats, efeats, edge_index, W_msg, b_msg, W_apply, b_apply):
    nfeats2 = nfeats.reshape(N, DIN)
    efeats2 = efeats.reshape(E, DE)
    src = edge_index[0].astype(jnp.int32)
    dst = edge_index[1].astype(jnp.int32)

    zn = jnp.zeros((NACC, DIN), jnp.float32)
    ze = jnp.zeros((NACC, DE), jnp.float32)

    snp, sep, degp = _scatter_kernel(
        nfeats2, efeats2,
        src.reshape(NS, NCHA, CHA), dst.reshape(NS, NCHA, CHA), zn, ze)

    nh3, hh3 = _dense(snp, sep, degp.T, nfeats2[None], W_msg,
                      b_msg[None], W_apply, b_apply[None])
    new_h = nh3.reshape(N, 1, DOUT)
    hh = hh3.reshape(N, DOUT)

    e_out = _edge_kernel(hh, src.reshape(NW, NCHC, CHC),
                         dst.reshape(NW, NCHC, CHC))
    return new_h, e_out.reshape(E, 1, DOUT)
